# baseline (speedup 1.0000x reference)
import functools

import jax
import jax.numpy as jnp
from jax import lax
from jax.experimental import pallas as pl
from jax.experimental.pallas import tpu as pltpu
from jax.experimental.pallas import tpu_sc as plsc

BLOCK = 8
KEEP = 4
NROW = 4096
NCOL = 4096
NC = 2
NS = 16
L = 16
NW = NC * NS
ROWS_W = NROW // NW
RCH = 4
OUTER = ROWS_W // RCH
GROUPS = NCOL // (BLOCK * L)

_mesh = plsc.VectorSubcoreMesh(core_axis_name="c", subcore_axis_name="s")


@functools.partial(
    pl.kernel,
    out_type=jax.ShapeDtypeStruct((NROW, NCOL), jnp.float32),
    mesh=_mesh,
    scratch_types=[
        pltpu.VMEM((RCH, NCOL), jnp.float32),
        pltpu.VMEM((RCH, NCOL), jnp.float32),
        pltpu.VMEM((RCH, NCOL), jnp.float32),
        pltpu.VMEM((RCH, NCOL), jnp.float32),
        pltpu.VMEM((RCH, NCOL), jnp.float32),
        pltpu.VMEM((RCH, NCOL), jnp.float32),
        pltpu.SemaphoreType.DMA,
        pltpu.SemaphoreType.DMA,
        pltpu.SemaphoreType.DMA,
        pltpu.SemaphoreType.DMA,
        pltpu.SemaphoreType.DMA,
        pltpu.SemaphoreType.DMA,
    ],
    compiler_params=pltpu.CompilerParams(needs_layout_passes=False),
)
def _sparsify_sc(x_hbm, s_hbm, o_hbm,
                 sA, sB, xA, xB, oA, oB,
                 ssA, ssB, sxA, sxB, soA, soB):
    wid = lax.axis_index("s") * NC + lax.axis_index("c")
    row0 = wid * ROWS_W
    lane = lax.iota(jnp.int32, L)
    offs = tuple(lane * BLOCK + j for j in range(BLOCK))
    step = jnp.full((L,), BLOCK * L, jnp.int32)
    zerof = jnp.full((L,), 0.0, jnp.float32)

    slots = ((sA, xA, oA, ssA, sxA, soA), (sB, xB, oB, ssB, sxB, soB))

    def ce(a, b):
        return jnp.minimum(a, b), jnp.maximum(a, b)

    def sort4(a, b, c, d):
        a, b = ce(a, b)
        c, d = ce(c, d)
        a, c = ce(a, c)
        b, d = ce(b, d)
        b, c = ce(b, c)
        return a, b, c, d

    def compute_chunk(sbuf, xbuf, obuf):
        for rr in range(RCH):
            rowv = jnp.full((L,), rr, jnp.int32)

            def inner(g, idx):
                s = [plsc.load_gather(sbuf, [rowv, idx[j]])
                     for j in range(BLOCK)]
                x = [plsc.load_gather(xbuf, [rowv, idx[j]])
                     for j in range(BLOCK)]
                a = sort4(s[0], s[1], s[2], s[3])
                b = sort4(s[4], s[5], s[6], s[7])
                hi = [jnp.maximum(a[i], b[3 - i]) for i in range(4)]
                t = jnp.minimum(jnp.minimum(hi[0], hi[1]),
                                jnp.minimum(hi[2], hi[3]))
                for j in range(BLOCK):
                    ov = jnp.where(s[j] >= t, x[j], zerof)
                    plsc.store_scatter(obuf, [rowv, idx[j]], ov)
                return tuple(idx[j] + step for j in range(BLOCK))

            lax.fori_loop(0, GROUPS, inner, offs)

    def start_in(i, sbuf, xbuf, ssem, xsem):
        r = row0 + i * RCH
        pltpu.async_copy(s_hbm.at[pl.ds(r, RCH)], sbuf, ssem)
        pltpu.async_copy(x_hbm.at[pl.ds(r, RCH)], xbuf, xsem)

    for b in range(2):
        sbuf, xbuf, obuf, ssem, xsem, osem = slots[b]
        start_in(b, sbuf, xbuf, ssem, xsem)

    def outer(io, carry):
        for b in range(2):
            i = io * 2 + b
            sbuf, xbuf, obuf, ssem, xsem, osem = slots[b]
            r = row0 + i * RCH
            pltpu.make_async_copy(s_hbm.at[pl.ds(0, RCH)], sbuf, ssem).wait()
            pltpu.make_async_copy(x_hbm.at[pl.ds(0, RCH)], xbuf, xsem).wait()
            @pl.when(i >= 2)
            def _():
                pltpu.make_async_copy(obuf, o_hbm.at[pl.ds(0, RCH)],
                                      osem).wait()

            compute_chunk(sbuf, xbuf, obuf)
            pltpu.async_copy(obuf, o_hbm.at[pl.ds(r, RCH)], osem)

            @pl.when(i + 2 < OUTER)
            def _():
                start_in(i + 2, sbuf, xbuf, ssem, xsem)
        return carry

    lax.fori_loop(0, OUTER // 2, outer, None)

    for b in range(2):
        sbuf, xbuf, obuf, ssem, xsem, osem = slots[b]
        pltpu.make_async_copy(obuf, o_hbm.at[pl.ds(0, RCH)], osem).wait()


def kernel(x, score):
    return _sparsify_sc(x, score)


# speedup vs baseline: 1.0013x; 1.0013x over previous
import functools

import jax
import jax.numpy as jnp
from jax import lax
from jax.experimental import pallas as pl
from jax.experimental.pallas import tpu as pltpu
from jax.experimental.pallas import tpu_sc as plsc

BLOCK = 8
KEEP = 4
NROW = 4096
NCOL = 4096
NC = 2
NS = 16
L = 16
NW = NC * NS
ROWS_W = NROW // NW
RCH = 4
OUTER = ROWS_W // RCH
GROUPS = NCOL // (BLOCK * L)

_mesh = plsc.VectorSubcoreMesh(core_axis_name="c", subcore_axis_name="s")


@functools.partial(
    pl.kernel,
    out_type=jax.ShapeDtypeStruct((NROW, NCOL), jnp.float32),
    mesh=_mesh,
    scratch_types=[
        pltpu.VMEM((RCH, NCOL), jnp.float32),
        pltpu.VMEM((RCH, NCOL), jnp.float32),
        pltpu.VMEM((RCH, NCOL), jnp.float32),
        pltpu.VMEM((RCH, NCOL), jnp.float32),
        pltpu.VMEM((RCH, NCOL), jnp.float32),
        pltpu.VMEM((RCH, NCOL), jnp.float32),
        pltpu.SemaphoreType.DMA,
        pltpu.SemaphoreType.DMA,
        pltpu.SemaphoreType.DMA,
        pltpu.SemaphoreType.DMA,
        pltpu.SemaphoreType.DMA,
        pltpu.SemaphoreType.DMA,
    ],
    compiler_params=pltpu.CompilerParams(
        needs_layout_passes=False,
        disable_bounds_checks=True,
        disable_semaphore_checks=True,
    ),
)
def _sparsify_sc(x_hbm, s_hbm, o_hbm,
                 sA, sB, xA, xB, oA, oB,
                 ssA, ssB, sxA, sxB, soA, soB):
    wid = lax.axis_index("s") * NC + lax.axis_index("c")
    row0 = wid * ROWS_W
    lane = lax.iota(jnp.int32, L)
    offs = tuple(lane * BLOCK + j for j in range(BLOCK))
    step = jnp.full((L,), BLOCK * L, jnp.int32)
    zerof = jnp.full((L,), 0.0, jnp.float32)

    slots = ((sA, xA, oA, ssA, sxA, soA), (sB, xB, oB, ssB, sxB, soB))

    def ce(a, b):
        return jnp.minimum(a, b), jnp.maximum(a, b)

    def sort4(a, b, c, d):
        a, b = ce(a, b)
        c, d = ce(c, d)
        a, c = ce(a, c)
        b, d = ce(b, d)
        b, c = ce(b, c)
        return a, b, c, d

    def compute_chunk(sbuf, xbuf, obuf):
        for rr in range(RCH):
            rowv = jnp.full((L,), rr, jnp.int32)

            def inner(g, idx):
                s = [plsc.load_gather(sbuf, [rowv, idx[j]])
                     for j in range(BLOCK)]
                x = [plsc.load_gather(xbuf, [rowv, idx[j]])
                     for j in range(BLOCK)]
                a = sort4(s[0], s[1], s[2], s[3])
                b = sort4(s[4], s[5], s[6], s[7])
                hi = [jnp.maximum(a[i], b[3 - i]) for i in range(4)]
                t = jnp.minimum(jnp.minimum(hi[0], hi[1]),
                                jnp.minimum(hi[2], hi[3]))
                for j in range(BLOCK):
                    ov = jnp.where(s[j] >= t, x[j], zerof)
                    plsc.store_scatter(obuf, [rowv, idx[j]], ov)
                return tuple(idx[j] + step for j in range(BLOCK))

            lax.fori_loop(0, GROUPS, inner, offs)

    def start_in(i, sbuf, xbuf, ssem, xsem):
        r = row0 + i * RCH
        pltpu.async_copy(s_hbm.at[pl.ds(r, RCH)], sbuf, ssem)
        pltpu.async_copy(x_hbm.at[pl.ds(r, RCH)], xbuf, xsem)

    for b in range(2):
        sbuf, xbuf, obuf, ssem, xsem, osem = slots[b]
        start_in(b, sbuf, xbuf, ssem, xsem)

    def outer(io, carry):
        for b in range(2):
            i = io * 2 + b
            sbuf, xbuf, obuf, ssem, xsem, osem = slots[b]
            r = row0 + i * RCH
            pltpu.make_async_copy(s_hbm.at[pl.ds(0, RCH)], sbuf, ssem).wait()
            pltpu.make_async_copy(x_hbm.at[pl.ds(0, RCH)], xbuf, xsem).wait()
            @pl.when(i >= 2)
            def _():
                pltpu.make_async_copy(obuf, o_hbm.at[pl.ds(0, RCH)],
                                      osem).wait()

            compute_chunk(sbuf, xbuf, obuf)
            pltpu.async_copy(obuf, o_hbm.at[pl.ds(r, RCH)], osem)

            @pl.when(i + 2 < OUTER)
            def _():
                start_in(i + 2, sbuf, xbuf, ssem, xsem)
        return carry

    lax.fori_loop(0, OUTER // 2, outer, None)

    for b in range(2):
        sbuf, xbuf, obuf, ssem, xsem, osem = slots[b]
        pltpu.make_async_copy(obuf, o_hbm.at[pl.ds(0, RCH)], osem).wait()


def kernel(x, score):
    return _sparsify_sc(x, score)


# speedup vs baseline: 1.0019x; 1.0006x over previous
import functools

import jax
import jax.numpy as jnp
from jax import lax
from jax.experimental import pallas as pl
from jax.experimental.pallas import tpu as pltpu
from jax.experimental.pallas import tpu_sc as plsc

BLOCK = 8
KEEP = 4
NROW = 4096
NCOL = 4096
NC = 2
NS = 16
L = 16
NW = NC * NS
ROWS_W = NROW // NW
RCH = 4
OUTER = ROWS_W // RCH
GROUPS = NCOL // (BLOCK * L)

_mesh = plsc.VectorSubcoreMesh(core_axis_name="c", subcore_axis_name="s")


@functools.partial(
    pl.kernel,
    out_type=jax.ShapeDtypeStruct((NROW, NCOL), jnp.float32),
    mesh=_mesh,
    scratch_types=[
        pltpu.VMEM((RCH, NCOL), jnp.float32),
        pltpu.VMEM((RCH, NCOL), jnp.float32),
        pltpu.VMEM((RCH, NCOL), jnp.float32),
        pltpu.VMEM((RCH, NCOL), jnp.float32),
        pltpu.VMEM((RCH, NCOL), jnp.float32),
        pltpu.VMEM((RCH, NCOL), jnp.float32),
        pltpu.SemaphoreType.DMA,
        pltpu.SemaphoreType.DMA,
        pltpu.SemaphoreType.DMA,
        pltpu.SemaphoreType.DMA,
        pltpu.SemaphoreType.DMA,
        pltpu.SemaphoreType.DMA,
    ],
    compiler_params=pltpu.CompilerParams(needs_layout_passes=False),
)
def _sparsify_sc(x_hbm, s_hbm, o_hbm,
                 sA, sB, xA, xB, oA, oB,
                 ssA, ssB, sxA, sxB, soA, soB):
    wid = lax.axis_index("s") * NC + lax.axis_index("c")
    row0 = wid * ROWS_W
    lane = lax.iota(jnp.int32, L)
    offs = tuple(lane * BLOCK + j for j in range(BLOCK))
    step = jnp.full((L,), BLOCK * L, jnp.int32)
    zerof = jnp.full((L,), 0.0, jnp.float32)

    slots = ((sA, xA, oA, ssA, sxA, soA), (sB, xB, oB, ssB, sxB, soB))

    def ce(a, b):
        return jnp.minimum(a, b), jnp.maximum(a, b)

    def sort4(a, b, c, d):
        a, b = ce(a, b)
        c, d = ce(c, d)
        a, c = ce(a, c)
        b, d = ce(b, d)
        b, c = ce(b, c)
        return a, b, c, d

    def compute_chunk(sbuf, xbuf, obuf):
        for rr in range(RCH):
            rowv = jnp.full((L,), rr, jnp.int32)

            def inner(g, idx):
                s = [plsc.load_gather(sbuf, [rowv, idx[j]])
                     for j in range(BLOCK)]
                x = [plsc.load_gather(xbuf, [rowv, idx[j]])
                     for j in range(BLOCK)]
                a = sort4(s[0], s[1], s[2], s[3])
                b = sort4(s[4], s[5], s[6], s[7])
                hi = [jnp.maximum(a[i], b[3 - i]) for i in range(4)]
                t = jnp.minimum(jnp.minimum(hi[0], hi[1]),
                                jnp.minimum(hi[2], hi[3]))
                for j in range(BLOCK):
                    ov = jnp.where(s[j] >= t, x[j], zerof)
                    plsc.store_scatter(obuf, [rowv, idx[j]], ov)
                return tuple(idx[j] + step for j in range(BLOCK))

            lax.fori_loop(0, GROUPS, inner, offs)

    def start_in(i, sbuf, xbuf, ssem, xsem):
        r = row0 + i * RCH
        pltpu.async_copy(s_hbm.at[pl.ds(r, RCH)], sbuf, ssem)
        pltpu.async_copy(x_hbm.at[pl.ds(r, RCH)], xbuf, xsem)

    for b in range(2):
        sbuf, xbuf, obuf, ssem, xsem, osem = slots[b]
        start_in(b, sbuf, xbuf, ssem, xsem)

    def outer(io, carry):
        for b in range(2):
            i = io * 2 + b
            sbuf, xbuf, obuf, ssem, xsem, osem = slots[b]
            r = row0 + i * RCH
            pltpu.make_async_copy(s_hbm.at[pl.ds(0, RCH)], sbuf, ssem).wait()
            pltpu.make_async_copy(x_hbm.at[pl.ds(0, RCH)], xbuf, xsem).wait()
            @pl.when(i >= 2)
            def _():
                pltpu.make_async_copy(obuf, o_hbm.at[pl.ds(0, RCH)],
                                      osem).wait()

            compute_chunk(sbuf, xbuf, obuf)
            pltpu.async_copy(obuf, o_hbm.at[pl.ds(r, RCH)], osem)

            @pl.when(i + 2 < OUTER)
            def _():
                start_in(i + 2, sbuf, xbuf, ssem, xsem)
        return carry

    lax.fori_loop(0, OUTER // 2, outer, None)

    for b in range(2):
        sbuf, xbuf, obuf, ssem, xsem, osem = slots[b]
        pltpu.make_async_copy(obuf, o_hbm.at[pl.ds(0, RCH)], osem).wait()


def kernel(x, score):
    return _sparsify_sc(x, score)
